# baseline (speedup 1.0000x reference)
import functools

import jax
import jax.numpy as jnp
from jax import lax
from jax.experimental import pallas as pl
from jax.experimental.pallas import tpu as pltpu
from jax.experimental.pallas import tpu_sc as plsc

EPS = 1e-05
_EPS = 1e-10
MAX_NORM = 1000.0

_IDX_CHUNK = 128
_BW = 8


@functools.lru_cache(maxsize=None)
def _make_sc_gather(B, D, NC, NS):
    NW = NC * NS
    BPW = B // NW
    NCH = BPW // _IDX_CHUNK
    mesh = plsc.VectorSubcoreMesh(core_axis_name="c", subcore_axis_name="s")
    f32 = jnp.float32

    @functools.partial(
        pl.kernel,
        mesh=mesh,
        compiler_params=pltpu.CompilerParams(use_tc_tiling_on_sc=False),
        out_type=[
            jax.ShapeDtypeStruct((B, D), f32),
            jax.ShapeDtypeStruct((B, D), f32),
            jax.ShapeDtypeStruct((B, D), f32),
            jax.ShapeDtypeStruct((B, _BW), f32),
        ],
        scratch_types=[
            pltpu.VMEM((BPW,), jnp.int32),
            pltpu.VMEM((BPW,), jnp.int32),
            pltpu.VMEM((BPW,), jnp.int32),
            pltpu.VMEM((BPW,), jnp.int32),
            pltpu.VMEM((BPW, D), f32),
            pltpu.VMEM((BPW, D), f32),
            pltpu.VMEM((BPW, D), f32),
            pltpu.VMEM((BPW, _BW), f32),
            pltpu.SemaphoreType.DMA,
        ],
    )
    def sc_gather(uid, lid, pid, pwid, utab, itab, btab,
                  u_out, l_out, p_out, b_out,
                  uidx, lidx, pidx, widx, urows, lrows, prows, brows, sem):
        wid = lax.axis_index("s") * NC + lax.axis_index("c")
        base = wid * BPW
        pltpu.sync_copy(uid.at[pl.ds(base, BPW)], uidx)
        pltpu.sync_copy(lid.at[pl.ds(base, BPW)], lidx)
        pltpu.sync_copy(pid.at[pl.ds(base, BPW)], pidx)
        pltpu.sync_copy(pwid.at[pl.ds(base, BPW)], widx)
        copies = []
        for j in range(NCH):
            sl = pl.ds(j * _IDX_CHUNK, _IDX_CHUNK)
            copies.append(pltpu.async_copy(utab.at[uidx.at[sl]], urows.at[sl], sem))
            copies.append(pltpu.async_copy(itab.at[lidx.at[sl]], lrows.at[sl], sem))
            copies.append(pltpu.async_copy(itab.at[pidx.at[sl]], prows.at[sl], sem))
            copies.append(pltpu.async_copy(btab.at[widx.at[sl]], brows.at[sl], sem))
        for c in copies:
            c.wait()
        pltpu.sync_copy(urows, u_out.at[pl.ds(base, BPW)])
        pltpu.sync_copy(lrows, l_out.at[pl.ds(base, BPW)])
        pltpu.sync_copy(prows, p_out.at[pl.ds(base, BPW)])
        pltpu.sync_copy(brows, b_out.at[pl.ds(base, BPW)])

    return sc_gather


def _exp_map_tail(v):
    v0 = v[:, :1]
    vr = v[:, 1:]
    ldv = -v0 * v0 + jnp.sum(vr * vr, axis=1, keepdims=True)
    nd = jnp.sqrt(jnp.clip(ldv + EPS, _EPS, None))
    t = jnp.minimum(nd, MAX_NORM)
    e = jnp.exp(t)
    sinh_t = 0.5 * (e - 1.0 / e)
    nr = (sinh_t / nd) * vr
    norms = jnp.sqrt(jnp.sum(nr * nr, axis=1, keepdims=True))
    factor = jnp.where(norms > MAX_NORM, MAX_NORM / jnp.maximum(norms, 1e-12), 1.0)
    nr = nr * factor
    first = jnp.sqrt(1.0 + jnp.sum(nr * nr, axis=1, keepdims=True))
    return first, nr


def _tc_math_body(u_ref, l_ref, p_ref, bw_ref, pm_ref, g_ref, o_ref):
    v = u_ref[...] + g_ref[...] + l_ref[...]
    a0, ar = _exp_map_tail(v)
    c0, cr = _exp_map_tail(p_ref[...])
    s = -a0 * c0 + jnp.sum(ar * cr, axis=1, keepdims=True)
    t2 = -s
    dist = jnp.log(t2 + jnp.sqrt(jnp.clip(t2 * t2 - 1.0, _EPS, None)))
    lanes = lax.broadcasted_iota(jnp.int32, bw_ref.shape, 1)
    b = jnp.sum(jnp.where(lanes == pm_ref[...], bw_ref[...], 0.0), axis=1,
                keepdims=True)
    o_ref[...] = -dist + b


def _tc_math(u, l, p, bw, pm, g, block):
    B, D = u.shape
    grid = (B // block,)
    spec_rows = pl.BlockSpec((block, D), lambda i: (i, 0))
    out = pl.pallas_call(
        _tc_math_body,
        grid=grid,
        in_specs=[
            spec_rows,
            spec_rows,
            spec_rows,
            pl.BlockSpec((block, _BW), lambda i: (i, 0)),
            pl.BlockSpec((block, 1), lambda i: (i, 0)),
            pl.BlockSpec((1, D), lambda i: (0, 0)),
        ],
        out_specs=pl.BlockSpec((block, 1), lambda i: (i, 0)),
        out_shape=jax.ShapeDtypeStruct((B, 1), jnp.float32),
    )(u, l, p, bw, pm, g)
    return out


def kernel(user_ids, last_items, pre_items, user_table, item_table,
           global_transition, item_biases):
    B = user_ids.shape[0]
    D = user_table.shape[1]
    V = item_biases.shape[0]
    info = plsc.get_sparse_core_info()
    sc_gather = _make_sc_gather(B, D, info.num_cores, info.num_subcores)
    pid = pre_items.astype(jnp.int32)
    u, l, p, bw = sc_gather(
        user_ids.astype(jnp.int32),
        last_items.astype(jnp.int32),
        pid,
        pid // _BW,
        user_table, item_table, item_biases.reshape(V // _BW, _BW))
    out = _tc_math(u, l, p, bw, (pid % _BW).reshape(B, 1), global_transition,
                   block=2048)
    return out.reshape(B)
